# baseline (speedup 1.0000x reference)
import functools

import numpy as np
import jax
import jax.numpy as jnp
from jax import lax
from jax.experimental import pallas as pl
from jax.experimental.pallas import tpu as pltpu
from jax.experimental.pallas import tpu_sc as plsc

_ALPHA = 0.7
_EPS = 1e-06
_RB = 128
_CH = 64

_ROTS = ((13, 15, 26, 6), (17, 29, 16, 24), (13, 15, 26, 6),
         (17, 29, 16, 24), (13, 15, 26, 6))
_KS = (0, 42, (0x1BD11BDA ^ 42) & 0xFFFFFFFF)
_INJ = tuple(((_KS[(g + 1) % 3]) & 0xFFFFFFFF,
              (_KS[(g + 2) % 3] + g + 1) & 0xFFFFFFFF) for g in range(5))


def _i32c(v):
    v &= 0xFFFFFFFF
    return jnp.int32(v - (1 << 32) if v >= (1 << 31) else v)


def _srl(x, r):
    return lax.shift_right_logical(x, jnp.int32(r))


def _bits23(hi, lo):
    x0 = hi
    x1 = lo + _i32c(_KS[1])
    for g in range(5):
        for r in _ROTS[g]:
            x0 = x0 + x1
            x1 = ((x1 << jnp.int32(r)) | _srl(x1, 32 - r)) ^ x0
        a, b = _INJ[g]
        x0 = x0 + _i32c(a)
        x1 = x1 + _i32c(b)
    return _srl(x0 ^ x1, 9)


def _sampler_body(nchunks, lmax_ref, rowlo_ref, rowhi_ref, wrapj_ref, lt_ref,
                  idx_ref, m_scr, i_scr):
    tiny = jnp.float32(np.finfo(np.float32).tiny)
    lm = lmax_ref[0:1, 0:1]
    rlo = rowlo_ref[0]
    rhi = rowhi_ref[0]
    wj = wrapj_ref[0]
    m_scr[...] = jnp.full((1, _RB), -jnp.inf, jnp.float32)
    i_scr[...] = jnp.zeros((1, _RB), jnp.int32)
    col0 = lax.broadcasted_iota(jnp.int32, (_CH, _RB), 0)

    def chunk(c, carry):
        jcol = col0 + c * _CH
        lo = rlo + jcol
        hi = rhi + (jcol >= wj).astype(jnp.int32)
        m = _bits23(hi, lo)
        cmax_m = jnp.max(m, axis=0, keepdims=True)
        cur = m_scr[...]
        t2 = jnp.exp(-jnp.exp(lm - cur))
        mt = (t2 * jnp.float32(8388608.0 * (1.0 - 1e-5))
              - jnp.float32(16.0)).astype(jnp.int32)
        anycand = jnp.max((cmax_m > mt).astype(jnp.int32))

        @pl.when(anycand > 0)
        def _():
            f = lax.bitcast_convert_type(m | jnp.int32(0x3F800000),
                                         jnp.float32) - jnp.float32(1.0)
            u = jnp.maximum(f, tiny)
            g = -jnp.log(-jnp.log(u))
            s = g + lt_ref[c].reshape(_CH, 1)
            smax = jnp.max(s, axis=0, keepdims=True)
            cand = jnp.where(s == smax, jcol, jnp.int32(0x7FFFFFFF))
            sidx = jnp.min(cand, axis=0, keepdims=True)
            better = smax > m_scr[...]
            i_scr[...] = jnp.where(better, sidx, i_scr[...])
            m_scr[...] = jnp.where(better, smax, m_scr[...])

        return carry

    lax.fori_loop(0, nchunks, chunk, 0)
    idx_ref[0] = i_scr[...]


def _make_sampler(B, N, interpret=False):
    assert B % _RB == 0 and N % _CH == 0
    nblocks = B // _RB
    nchunks = N // _CH
    return pl.pallas_call(
        functools.partial(_sampler_body, nchunks),
        grid=(nblocks,),
        in_specs=[
            pl.BlockSpec((1, 1), lambda b: (0, 0)),
            pl.BlockSpec((1, 1, _RB), lambda b: (b, 0, 0)),
            pl.BlockSpec((1, 1, _RB), lambda b: (b, 0, 0)),
            pl.BlockSpec((1, 1, _RB), lambda b: (b, 0, 0)),
            pl.BlockSpec((nchunks, _CH), lambda b: (0, 0)),
        ],
        out_specs=pl.BlockSpec((1, 1, _RB), lambda b: (b, 0, 0)),
        out_shape=jax.ShapeDtypeStruct((nblocks, 1, _RB), jnp.int32),
        scratch_shapes=[
            pltpu.VMEM((1, _RB), jnp.float32),
            pltpu.VMEM((1, _RB), jnp.int32),
        ],
        interpret=interpret,
    )


def _row_counters(B, N):
    i = jnp.arange(B, dtype=jnp.int32)
    rowlo = i * jnp.int32(N)
    k = 1
    hi = jnp.zeros((B,), jnp.int32)
    while (k << 32) < B * N:
        hi = hi + (i >= ((k << 32) + N - 1) // N).astype(jnp.int32)
        k += 1
    neg = -rowlo
    wrapj = jnp.where((neg > 0) & (neg < N), neg, jnp.int32(2 * N))
    nb = B // _RB
    return (rowlo.reshape(nb, 1, _RB), hi.reshape(nb, 1, _RB),
            wrapj.reshape(nb, 1, _RB))


def _sample_indices(weights, B, interpret=False):
    N = weights.shape[0]
    w = weights + _EPS
    p = jnp.power(w, _ALPHA)
    probabilities = p / jnp.sum(p)
    logits = jnp.log(probabilities)
    lt = logits.reshape(N // _CH, _CH)
    lmax = jnp.max(logits).reshape(1, 1)
    rowlo, rowhi, wrapj = _row_counters(B, N)
    idx3 = _make_sampler(B, N, interpret=interpret)(lmax, rowlo, rowhi,
                                                    wrapj, lt)
    return idx3.reshape(B)


def _make_gather(V, D, B):
    info = plsc.get_sparse_core_info()
    nw = info.num_cores * info.num_subcores
    assert B % (8 * nw) == 0 and D % info.num_lanes == 0
    bw = B // nw
    mesh = plsc.VectorSubcoreMesh(core_axis_name="c", subcore_axis_name="s")

    @functools.partial(
        pl.kernel, mesh=mesh,
        out_type=jax.ShapeDtypeStruct((B, D), jnp.float32),
        scratch_types=[
            pltpu.VMEM((bw,), jnp.int32),
            pltpu.VMEM((bw, D), jnp.float32),
            pltpu.SemaphoreType.DMA,
        ],
    )
    def k(table_hbm, idx_hbm, out_hbm, idx_v, rows_v, sem):
        wid = lax.axis_index("s") * info.num_cores + lax.axis_index("c")
        base = wid * bw
        pltpu.sync_copy(idx_hbm.at[pl.ds(base, bw)], idx_v)
        pltpu.async_copy(table_hbm.at[idx_v], rows_v, sem).wait()
        pltpu.sync_copy(rows_v, out_hbm.at[pl.ds(base, bw)])

    return k


def kernel(states, actions, rewards, next_states, terminals, weights):
    N = states.shape[0]
    B = 16384
    idx = _sample_indices(weights, B)
    table = jnp.concatenate(
        [states, actions, rewards, next_states, terminals,
         jnp.zeros((N, 54), jnp.float32)], axis=1)
    out = _make_gather(N, 128, B)(table, idx)
    return (out[:, 0:32], out[:, 32:40], out[:, 40:41],
            out[:, 41:73], out[:, 73:74])
